# baseline (speedup 1.0000x reference)
import functools
import jax
import jax.numpy as jnp
from jax import lax
from jax.experimental import pallas as pl
from jax.experimental.pallas import tpu as pltpu
from jax.experimental.pallas import tpu_sc as plsc

WORDLEN = 100000
EMB = 64
CTX = 20
HID = 128
BK = 2048
G = 7
NJ = 7
PAD = G * NJ * BK
NEG = -jnp.inf
NC_SC = 2


_mesh = plsc.VectorSubcoreMesh(core_axis_name="c", subcore_axis_name="s")


@functools.partial(
    pl.kernel, mesh=_mesh,
    out_type=jax.ShapeDtypeStruct((CTX + 4, 2 * EMB), jnp.float32),
    scratch_types=[
        pltpu.VMEM((24,), jnp.int32),
        pltpu.VMEM((CTX + 4, 2 * EMB), jnp.float32),
        pltpu.SemaphoreType.DMA,
    ],
)
def _sc_emb(table2_hbm, idx_hbm, out_hbm, idx_v, rows_v, sem):
    wid = lax.axis_index("s") * NC_SC + lax.axis_index("c")

    @pl.when(wid == 0)
    def _():
        pltpu.sync_copy(idx_hbm, idx_v)
        pltpu.async_copy(table2_hbm.at[idx_v], rows_v, sem).wait()
        pltpu.sync_copy(rows_v, out_hbm)


def _fused(x_ref, embp_ref, w1_ref, b1_ref, *refs):
    w2_blks = refs[:G]
    b2_ref, out_ref, h_ref, m_ref = refs[G:]
    j = pl.program_id(0)

    @pl.when(j == 0)
    def _compute_h():
        acc = b1_ref[...]
        for i in range(CTX):
            pair = embp_ref[i:i + 1, :]
            even = lax.rem(x_ref[i], 2) == 0
            row = jnp.where(even, pair[:, :EMB], pair[:, EMB:])
            acc = acc + jnp.dot(row,
                                w1_ref[i * EMB:(i + 1) * EMB, :],
                                preferred_element_type=jnp.float32)
        h_ref[...] = jnp.maximum(acc, 0.0)
        m_ref[...] = jnp.full((1, BK), NEG, jnp.float32)

    h = h_ref[...]
    m = m_ref[...]
    for g in range(G):
        bidx = g * NJ + j
        logits = jnp.dot(h, w2_blks[g][...],
                         preferred_element_type=jnp.float32)
        logits = logits + b2_ref[:, pl.ds(bidx * BK, BK)]
        col = lax.broadcasted_iota(jnp.int32, (1, BK), 1) + bidx * BK
        logits = jnp.where(col < WORDLEN, logits, NEG)
        out_ref[:, pl.ds(bidx * BK, BK)] = logits
        m = jnp.maximum(m, logits)
    m_ref[...] = m

    @pl.when(j == NJ - 1)
    def _finalize():
        mx = jnp.max(m_ref[...])
        lo = out_ref[...]
        s = jnp.sum(jnp.exp(lo - mx))
        out_ref[...] = lo - (mx + jnp.log(s))


def kernel(x, table, W1, b1, W2, b2):
    b1r = b1.reshape(1, HID)
    b2p = jnp.pad(b2, (0, PAD - WORDLEN)).reshape(1, PAD)

    xp = jnp.pad(x, (0, 4)).astype(jnp.int32)
    table2 = table.reshape(WORDLEN // 2, 2 * EMB)
    embp = _sc_emb(table2, xp // 2)

    w2_specs = [
        pl.BlockSpec((HID, BK), lambda j, xr, g=g: (0, g * NJ + j))
        for g in range(G)
    ]
    grid_spec = pltpu.PrefetchScalarGridSpec(
        num_scalar_prefetch=1,
        grid=(NJ,),
        in_specs=[
            pl.BlockSpec((CTX + 4, 2 * EMB), lambda j, xr: (0, 0)),
            pl.BlockSpec((HID * 10, HID), lambda j, xr: (0, 0)),
            pl.BlockSpec((1, HID), lambda j, xr: (0, 0)),
            *w2_specs,
            pl.BlockSpec((1, PAD), lambda j, xr: (0, 0)),
        ],
        out_specs=pl.BlockSpec((1, PAD), lambda j, xr: (0, 0)),
        scratch_shapes=[
            pltpu.VMEM((1, HID), jnp.float32),
            pltpu.VMEM((1, BK), jnp.float32),
        ],
    )
    out = pl.pallas_call(
        _fused,
        grid_spec=grid_spec,
        out_shape=jax.ShapeDtypeStruct((1, PAD), jnp.float32),
    )(x, embp, W1, b1r, *([W2] * G), b2p)
    return out[:, :WORDLEN]
